# baseline (speedup 1.0000x reference)
import dataclasses
import functools

import jax
import jax.numpy as jnp
from jax import lax
from jax.experimental import pallas as pl
from jax.experimental.pallas import tpu as pltpu
from jax.experimental.pallas import tpu_sc as plsc

BATCH = 16384
EMBED_DIM = 64
ROW = 2 * EMBED_DIM
NC = 2
NS = 16
NW = NC * NS
B_PER_W = BATCH // NW
CHUNK = 256
LANES = 16


def kernel(gender_idx, age_idx, occupation_idx, area_idx, u_id,
           W_gender, W_age, W_occupation, W_area):
    del u_id
    g = gender_idx.astype(jnp.int32)
    a = age_idx.astype(jnp.int32)
    o = occupation_idx.astype(jnp.int32)
    ar = area_idx.astype(jnp.int32)

    W_ga = jnp.concatenate(
        [jnp.repeat(W_gender, 7, axis=0), jnp.tile(W_age, (2, 1))], axis=1)
    W_occ128 = jnp.pad(W_occupation, ((0, 0), (0, EMBED_DIM)))
    X_area = W_area.reshape(-1, ROW)

    ga_idx = g * 7 + a
    ar2 = ar // 2
    harr = (ar % 2) * EMBED_DIM
    rows2 = jnp.arange(BATCH, dtype=jnp.int32) * 2
    even_rows = rows2
    odd_rows = rows2 + 1

    mesh = plsc.VectorSubcoreMesh(core_axis_name="c", subcore_axis_name="s")
    cp = pltpu.CompilerParams()
    if "needs_layout_passes" in pltpu.CompilerParams.__dataclass_fields__:
        cp = dataclasses.replace(cp, needs_layout_passes=False)

    @functools.partial(
        pl.kernel,
        mesh=mesh,
        compiler_params=cp,
        out_type=jax.ShapeDtypeStruct((2 * BATCH, ROW), jnp.float32),
        scratch_types=[
            pltpu.VMEM((CHUNK,), jnp.int32),
            pltpu.VMEM((CHUNK,), jnp.int32),
            pltpu.VMEM((CHUNK, ROW), jnp.float32),
            pltpu.VMEM((CHUNK, ROW), jnp.float32),
            pltpu.VMEM((CHUNK,), jnp.int32),
        ],
    )
    def emb_kernel(ga_hbm, o_hbm, ar2_hbm, h_hbm, er_hbm, or_hbm,
                   wga_hbm, wocc_hbm, xar_hbm,
                   out_hbm, sidx_v, didx_v, tmp_v, area_v, h_v):
        wid = lax.axis_index("s") * NC + lax.axis_index("c")
        for chunk in range(B_PER_W // CHUNK):
            base = wid * B_PER_W + chunk * CHUNK
            pltpu.sync_copy(ga_hbm.at[pl.ds(base, CHUNK)], sidx_v)
            pltpu.sync_copy(wga_hbm.at[sidx_v], tmp_v)
            pltpu.sync_copy(er_hbm.at[pl.ds(base, CHUNK)], didx_v)
            pltpu.sync_copy(tmp_v, out_hbm.at[didx_v])
            pltpu.sync_copy(o_hbm.at[pl.ds(base, CHUNK)], sidx_v)
            pltpu.sync_copy(wocc_hbm.at[sidx_v], tmp_v)
            pltpu.sync_copy(ar2_hbm.at[pl.ds(base, CHUNK)], sidx_v)
            pltpu.sync_copy(xar_hbm.at[sidx_v], area_v)
            pltpu.sync_copy(h_hbm.at[pl.ds(base, CHUNK)], h_v)

            lanes_iota = lax.iota(jnp.int32, LANES)

            @pl.loop(0, CHUNK, step=LANES)
            def _(j0):
                rowv = lanes_iota + j0
                hv = h_v[pl.ds(j0, LANES)]

                @pl.loop(0, EMBED_DIM)
                def _(c):
                    vals = plsc.load_gather(area_v, [rowv, hv + c])
                    plsc.store_scatter(
                        tmp_v, [rowv, lanes_iota * 0 + (EMBED_DIM + c)], vals)

            pltpu.sync_copy(or_hbm.at[pl.ds(base, CHUNK)], didx_v)
            pltpu.sync_copy(tmp_v, out_hbm.at[didx_v])

    out2 = emb_kernel(ga_idx, o, ar2, harr, even_rows, odd_rows,
                      W_ga, W_occ128, X_area)
    return out2.reshape(BATCH, 2 * ROW)


# speedup vs baseline: 1.1419x; 1.1419x over previous
import dataclasses
import functools

import jax
import jax.numpy as jnp
from jax import lax
from jax.experimental import pallas as pl
from jax.experimental.pallas import tpu as pltpu
from jax.experimental.pallas import tpu_sc as plsc

BATCH = 16384
EMBED_DIM = 64
ROW = 2 * EMBED_DIM
NC = 2
NS = 16
NW = NC * NS
B_PER_W = BATCH // NW
CHUNK = 128
NCH = B_PER_W // CHUNK
LANES = 16


def kernel(gender_idx, age_idx, occupation_idx, area_idx, u_id,
           W_gender, W_age, W_occupation, W_area):
    del u_id
    g = gender_idx.astype(jnp.int32)
    a = age_idx.astype(jnp.int32)
    o = occupation_idx.astype(jnp.int32)
    ar = area_idx.astype(jnp.int32)

    W_ga = jnp.concatenate(
        [jnp.repeat(W_gender, 7, axis=0), jnp.tile(W_age, (2, 1))], axis=1)
    W_occ128 = jnp.pad(W_occupation, ((0, 0), (0, EMBED_DIM)))
    X_area = W_area.reshape(-1, ROW)

    mesh = plsc.VectorSubcoreMesh(core_axis_name="c", subcore_axis_name="s")
    cp = pltpu.CompilerParams()
    if "needs_layout_passes" in pltpu.CompilerParams.__dataclass_fields__:
        cp = dataclasses.replace(cp, needs_layout_passes=False)

    idx_slot = [pltpu.VMEM((CHUNK,), jnp.int32)] * 4
    big_slot = [pltpu.VMEM((CHUNK, ROW), jnp.float32)] * 3
    aux_slot = [pltpu.VMEM((CHUNK,), jnp.int32)] * 3
    sem_slot = [pltpu.SemaphoreType.DMA] * 4

    @functools.partial(
        pl.kernel,
        mesh=mesh,
        compiler_params=cp,
        out_type=jax.ShapeDtypeStruct((2 * BATCH, ROW), jnp.float32),
        scratch_types=(idx_slot + big_slot + aux_slot) * 2 + sem_slot * 2,
    )
    def emb_kernel(g_hbm, a_hbm, o_hbm, ar_hbm,
                   wga_hbm, wocc_hbm, xar_hbm, out_hbm,
                   gv0, av0, ov0, arv0, tmpe0, tmpo0, area0, hv0, de0, do0,
                   gv1, av1, ov1, arv1, tmpe1, tmpo1, area1, hv1, de1, do1,
                   semi0, seme0, semo0, sems0,
                   semi1, seme1, semo1, sems1):
        slots = (
            dict(gv=gv0, av=av0, ov=ov0, arv=arv0, tmpe=tmpe0, tmpo=tmpo0,
                 area=area0, hv=hv0, de=de0, do=do0,
                 semi=semi0, seme=seme0, semo=semo0, sems=sems0),
            dict(gv=gv1, av=av1, ov=ov1, arv=arv1, tmpe=tmpe1, tmpo=tmpo1,
                 area=area1, hv=hv1, de=de1, do=do1,
                 semi=semi1, seme=seme1, semo=semo1, sems=sems1),
        )
        wid = lax.axis_index("s") * NC + lax.axis_index("c")
        iot = lax.iota(jnp.int32, LANES)

        def issue_idx_loads(sl, base):
            return [pltpu.async_copy(src.at[pl.ds(base, CHUNK)], dst,
                                     sl["semi"])
                    for src, dst in ((g_hbm, sl["gv"]), (a_hbm, sl["av"]),
                                     (o_hbm, sl["ov"]), (ar_hbm, sl["arv"]))]

        def prep(sl, base):
            for t in range(CHUNK // LANES):
                s = pl.ds(t * LANES, LANES)
                sl["gv"].at[s][...] = sl["gv"].at[s][...] * 7 + \
                    sl["av"].at[s][...]
                ar16 = sl["arv"].at[s][...]
                sl["hv"].at[s][...] = lax.shift_left(
                    jnp.bitwise_and(ar16, 1), EMBED_DIM.bit_length() - 1)
                sl["arv"].at[s][...] = lax.shift_right_logical(ar16, 1)
                even = iot * 2 + (2 * base + 2 * t * LANES)
                sl["de"].at[s][...] = even
                sl["do"].at[s][...] = even + 1

        def select_half(sl):
            @pl.loop(0, CHUNK, step=LANES)
            def _(j0):
                rowv = iot + j0
                hvv = sl["hv"].at[pl.ds(j0, LANES)][...]
                for c in range(EMBED_DIM):
                    vals = plsc.load_gather(sl["area"], [rowv, hvv + c])
                    plsc.store_scatter(
                        sl["tmpo"], [rowv, iot * 0 + (EMBED_DIM + c)], vals)

        pending_idx = {0: issue_idx_loads(slots[0], wid * B_PER_W), 1: None}
        pending_scat = {0: [], 1: []}
        for c in range(NCH):
            s = c % 2
            sl = slots[s]
            base = wid * B_PER_W + c * CHUNK
            for h in pending_idx[s]:
                h.wait()
            prep(sl, base)
            ge = pltpu.async_copy(wga_hbm.at[sl["gv"]], sl["tmpe"], sl["seme"])
            go = pltpu.async_copy(wocc_hbm.at[sl["ov"]], sl["tmpo"], sl["semo"])
            ga = pltpu.async_copy(xar_hbm.at[sl["arv"]], sl["area"], sl["semo"])
            if c + 1 < NCH:
                nsl = slots[1 - s]
                for h in pending_scat[1 - s]:
                    h.wait()
                pending_scat[1 - s] = []
                pending_idx[1 - s] = issue_idx_loads(nsl, base + CHUNK)
            ge.wait()
            se = pltpu.async_copy(sl["tmpe"], out_hbm.at[sl["de"]], sl["sems"])
            go.wait()
            ga.wait()
            select_half(sl)
            so = pltpu.async_copy(sl["tmpo"], out_hbm.at[sl["do"]], sl["sems"])
            pending_scat[s] = [se, so]
        for s in (0, 1):
            for h in pending_scat[s]:
                h.wait()

    out2 = emb_kernel(g, a, o, ar, W_ga, W_occ128, X_area)
    return out2.reshape(BATCH, 2 * ROW)


# speedup vs baseline: 1.8063x; 1.5818x over previous
import dataclasses
import functools

import jax
import jax.numpy as jnp
from jax import lax
from jax.experimental import pallas as pl
from jax.experimental.pallas import tpu as pltpu
from jax.experimental.pallas import tpu_sc as plsc

BATCH = 16384
EMBED_DIM = 64
OUT_W = 4 * EMBED_DIM
ROW = 2 * EMBED_DIM
NC = 2
NS = 16
NW = NC * NS
B_PER_W = BATCH // NW
CHUNK = 128
NCH = B_PER_W // CHUNK
LANES = 16


def kernel(gender_idx, age_idx, occupation_idx, area_idx, u_id,
           W_gender, W_age, W_occupation, W_area):
    del u_id
    g = gender_idx.astype(jnp.int32)
    a = age_idx.astype(jnp.int32)
    o = occupation_idx.astype(jnp.int32)
    ar = area_idx.astype(jnp.int32)

    n_gao = 2 * 7 * 21
    cid = jnp.arange(n_gao, dtype=jnp.int32)
    W_gao = jnp.concatenate(
        [W_gender[cid // (7 * 21)], W_age[(cid // 21) % 7],
         W_occupation[cid % 21],
         jnp.zeros((n_gao, EMBED_DIM), jnp.float32)], axis=1)
    X_area = W_area.reshape(-1, ROW)

    mesh = plsc.VectorSubcoreMesh(core_axis_name="c", subcore_axis_name="s")
    cp = pltpu.CompilerParams()
    if "needs_layout_passes" in pltpu.CompilerParams.__dataclass_fields__:
        cp = dataclasses.replace(cp, needs_layout_passes=False)

    @functools.partial(
        pl.kernel,
        mesh=mesh,
        compiler_params=cp,
        out_type=jax.ShapeDtypeStruct((BATCH, OUT_W), jnp.float32),
        scratch_types=[
            pltpu.VMEM((B_PER_W,), jnp.int32),
            pltpu.VMEM((B_PER_W,), jnp.int32),
            pltpu.VMEM((B_PER_W,), jnp.int32),
            pltpu.VMEM((B_PER_W,), jnp.int32),
            pltpu.VMEM((CHUNK, OUT_W), jnp.float32),
            pltpu.VMEM((CHUNK, OUT_W), jnp.float32),
            pltpu.VMEM((CHUNK, ROW), jnp.float32),
            pltpu.VMEM((CHUNK, ROW), jnp.float32),
            pltpu.SemaphoreType.DMA,
            pltpu.SemaphoreType.DMA,
            pltpu.SemaphoreType.DMA,
            pltpu.SemaphoreType.DMA,
            pltpu.SemaphoreType.DMA,
        ],
    )
    def emb_kernel(g_hbm, a_hbm, o_hbm, ar_hbm, wgao_hbm, xar_hbm, out_hbm,
                   gaov, tv, ov, hv, cat0, cat1, area0, area1,
                   semi, semg0, semg1, sems0, sems1):
        cats = (cat0, cat1)
        areas = (area0, area1)
        semg = (semg0, semg1)
        sems = (sems0, sems1)
        wid = lax.axis_index("s") * NC + lax.axis_index("c")
        base = wid * B_PER_W
        iot = lax.iota(jnp.int32, LANES)

        loads = [pltpu.async_copy(src.at[pl.ds(base, B_PER_W)], dst, semi)
                 for src, dst in ((g_hbm, gaov), (a_hbm, tv),
                                 (o_hbm, ov), (ar_hbm, hv))]
        for h in loads:
            h.wait()
        for t in range(B_PER_W // LANES):
            s = pl.ds(t * LANES, LANES)
            gaov.at[s][...] = (gaov.at[s][...] * 7 + tv.at[s][...]) * 21 + \
                ov.at[s][...]
            ar16 = hv.at[s][...]
            tv.at[s][...] = lax.shift_right_logical(ar16, 1)
            hv.at[s][...] = lax.shift_left(jnp.bitwise_and(ar16, 1), 6)

        def issue_gathers(c):
            s = c % 2
            off = pl.ds(c * CHUNK, CHUNK)
            return [
                pltpu.async_copy(wgao_hbm.at[gaov.at[off]], cats[s], semg[s]),
                pltpu.async_copy(xar_hbm.at[tv.at[off]], areas[s], semg[s]),
            ]

        def select_half(c):
            s = c % 2
            @pl.loop(0, CHUNK, step=LANES)
            def _(j0):
                rowv = iot + j0
                hvv = hv.at[pl.ds(c * CHUNK + j0, LANES)][...]
                colv = iot * 0 + (3 * EMBED_DIM)
                for cc in range(EMBED_DIM):
                    vals = plsc.load_gather(areas[s], [rowv, hvv + cc])
                    plsc.store_scatter(cats[s], [rowv, colv + cc], vals)

        pend_g = {0: issue_gathers(0), 1: None}
        pend_s = {0: None, 1: None}
        for c in range(NCH):
            s = c % 2
            for h in pend_g[s]:
                h.wait()
            if c + 1 < NCH:
                if pend_s[1 - s] is not None:
                    pend_s[1 - s].wait()
                    pend_s[1 - s] = None
                pend_g[1 - s] = issue_gathers(c + 1)
            select_half(c)
            pend_s[s] = pltpu.async_copy(
                cats[s], out_hbm.at[pl.ds(base + c * CHUNK, CHUNK)], sems[s])
        for s in (0, 1):
            if pend_s[s] is not None:
                pend_s[s].wait()

    return emb_kernel(g, a, o, ar, W_gao, X_area)
